# baseline (speedup 1.0000x reference)
import functools

import jax
import jax.numpy as jnp
from jax import lax
from jax.experimental import pallas as pl
from jax.experimental.pallas import tpu as pltpu
from jax.experimental.pallas import tpu_sc as plsc

ORDER = 16
SUBSAMPLING_FACTOR = 2
NUM_FEATURES = 64
SUB_ORDER = ORDER // SUBSAMPLING_FACTOR

NC = 2
NS = 16
NW = NC * NS
ROWS_PER_W = 2


def _make_sc_copy(n_out_rows, row):
    mesh = plsc.VectorSubcoreMesh(core_axis_name="c", subcore_axis_name="s")

    @functools.partial(
        pl.kernel,
        mesh=mesh,
        out_type=jax.ShapeDtypeStruct((n_out_rows, row), jnp.float32),
        scratch_types=[pltpu.SemaphoreType.DMA, pltpu.SemaphoreType.DMA],
    )
    def k(x_hbm, out_hbm, sem0, sem1):
        wid = lax.axis_index("s") * NC + lax.axis_index("c")
        copies = []
        for r in range(ROWS_PER_W):
            orow = wid * ROWS_PER_W + r
            irow = (orow // SUB_ORDER) * ORDER + (orow % SUB_ORDER) * SUBSAMPLING_FACTOR
            sem = (sem0, sem1)[r]
            copies.append(
                pltpu.make_async_copy(x_hbm.at[irow], out_hbm.at[orow], sem)
            )
        for c in copies:
            c.start()
        for c in copies:
            c.wait()

    return k


def kernel(x):
    B, C, H, W = x.shape
    row = NUM_FEATURES * H * W
    xr = x.reshape(B * ORDER, row)
    out = _make_sc_copy(B * SUB_ORDER, row)(xr)
    return out.reshape(B, SUB_ORDER * NUM_FEATURES, H, W)


# speedup vs baseline: 4.5434x; 4.5434x over previous
import functools

import jax
import jax.numpy as jnp
from jax import lax
from jax.experimental import pallas as pl
from jax.experimental.pallas import tpu as pltpu
from jax.experimental.pallas import tpu_sc as plsc

ORDER = 16
SUBSAMPLING_FACTOR = 2
NUM_FEATURES = 64
SUB_ORDER = ORDER // SUBSAMPLING_FACTOR

NC = 2
NS = 16
NW = NC * NS
ROWS_PER_W = 2


CHUNK = 32768
NBUF = 3


def _make_sc_copy(n_out_rows, row):
    mesh = plsc.VectorSubcoreMesh(core_axis_name="c", subcore_axis_name="s")
    chunks_per_row = row // CHUNK
    n_chunks = ROWS_PER_W * chunks_per_row

    @functools.partial(
        pl.kernel,
        mesh=mesh,
        out_type=jax.ShapeDtypeStruct((n_out_rows, row), jnp.float32),
        scratch_types=[pltpu.VMEM((NBUF, CHUNK), jnp.float32)]
        + [pltpu.SemaphoreType.DMA] * (2 * NBUF),
    )
    def k(x_hbm, out_hbm, buf, *sems):
        sin = sems[:NBUF]
        sout = sems[NBUF:]
        wid = lax.axis_index("s") * NC + lax.axis_index("c")

        def mk(t):
            orow = wid * ROWS_PER_W + t // chunks_per_row
            irow = (orow // SUB_ORDER) * ORDER + (orow % SUB_ORDER) * SUBSAMPLING_FACTOR
            off = (t % chunks_per_row) * CHUNK
            b = t % NBUF
            cin = pltpu.make_async_copy(
                x_hbm.at[pl.ds(irow, 1), pl.ds(off, CHUNK)],
                buf.at[pl.ds(b, 1), :],
                sin[b],
            )
            cout = pltpu.make_async_copy(
                buf.at[pl.ds(b, 1), :],
                out_hbm.at[pl.ds(orow, 1), pl.ds(off, CHUNK)],
                sout[b],
            )
            return cin, cout

        copies = [mk(t) for t in range(n_chunks)]
        copies[0][0].start()
        for t in range(n_chunks):
            if t + 1 < n_chunks:
                if t + 1 >= NBUF:
                    copies[t + 1 - NBUF][1].wait()
                copies[t + 1][0].start()
            copies[t][0].wait()
            copies[t][1].start()
        for t in range(max(0, n_chunks - NBUF), n_chunks):
            copies[t][1].wait()

    return k


def kernel(x):
    B, C, H, W = x.shape
    row = NUM_FEATURES * H * W
    xr = x.reshape(B * ORDER, row)
    out = _make_sc_copy(B * SUB_ORDER, row)(xr)
    return out.reshape(B, SUB_ORDER * NUM_FEATURES, H, W)


# speedup vs baseline: 4.5581x; 1.0032x over previous
import functools

import jax
import jax.numpy as jnp
from jax import lax
from jax.experimental import pallas as pl
from jax.experimental.pallas import tpu as pltpu
from jax.experimental.pallas import tpu_sc as plsc

ORDER = 16
SUBSAMPLING_FACTOR = 2
NUM_FEATURES = 64
SUB_ORDER = ORDER // SUBSAMPLING_FACTOR

NC = 2
NS = 16
NW = NC * NS
ROWS_PER_W = 2


CHUNK = 32768
NBUF = 3


def _make_sc_copy(n_out_rows, row):
    mesh = plsc.VectorSubcoreMesh(core_axis_name="c", subcore_axis_name="s")
    chunks_per_row = row // CHUNK
    n_chunks = ROWS_PER_W * chunks_per_row

    @functools.partial(
        pl.kernel,
        mesh=mesh,
        out_type=jax.ShapeDtypeStruct((n_out_rows, row), jnp.float32),
        scratch_types=[pltpu.VMEM_SHARED((NS, NBUF, CHUNK), jnp.float32)]
        + [pltpu.SemaphoreType.DMA] * (2 * NBUF),
    )
    def k(x_hbm, out_hbm, sbuf, *sems):
        sin = sems[:NBUF]
        sout = sems[NBUF:]
        sid = lax.axis_index("s")
        buf = sbuf.at[sid]
        wid = sid * NC + lax.axis_index("c")

        def mk(t):
            orow = wid * ROWS_PER_W + t // chunks_per_row
            irow = (orow // SUB_ORDER) * ORDER + (orow % SUB_ORDER) * SUBSAMPLING_FACTOR
            off = (t % chunks_per_row) * CHUNK
            b = t % NBUF
            cin = pltpu.make_async_copy(
                x_hbm.at[pl.ds(irow, 1), pl.ds(off, CHUNK)],
                buf.at[pl.ds(b, 1), :],
                sin[b],
            )
            cout = pltpu.make_async_copy(
                buf.at[pl.ds(b, 1), :],
                out_hbm.at[pl.ds(orow, 1), pl.ds(off, CHUNK)],
                sout[b],
            )
            return cin, cout

        copies = [mk(t) for t in range(n_chunks)]
        copies[0][0].start()
        for t in range(n_chunks):
            if t + 1 < n_chunks:
                if t + 1 >= NBUF:
                    copies[t + 1 - NBUF][1].wait()
                copies[t + 1][0].start()
            copies[t][0].wait()
            copies[t][1].start()
        for t in range(max(0, n_chunks - NBUF), n_chunks):
            copies[t][1].wait()

    return k


def kernel(x):
    B, C, H, W = x.shape
    row = NUM_FEATURES * H * W
    xr = x.reshape(B * ORDER, row)
    out = _make_sc_copy(B * SUB_ORDER, row)(xr)
    return out.reshape(B, SUB_ORDER * NUM_FEATURES, H, W)
